# baseline (speedup 1.0000x reference)
import functools

import jax
import jax.numpy as jnp
from jax import lax
from jax.experimental import pallas as pl
from jax.experimental.pallas import tpu as pltpu
from jax.experimental.pallas import tpu_sc as plsc

_INFO = plsc.get_sparse_core_info()
_NC = _INFO.num_cores
_NS = _INFO.num_subcores
_NW = _NC * _NS

_IW = 128
_G = 2
_NBUF = 4


def _gather_impl(table, idx2d, n_rows, embed):
    n_per_w = n_rows // _NW
    iw_per_w = n_per_w // _IW
    blk = _G * _IW
    n_blocks = n_per_w // blk
    assert n_blocks % _NBUF == 0 and n_blocks * blk == n_per_w

    mesh = plsc.VectorSubcoreMesh(core_axis_name="c", subcore_axis_name="s")

    @functools.partial(
        pl.kernel,
        mesh=mesh,
        compiler_params=pltpu.CompilerParams(use_tc_tiling_on_sc=False),
        out_type=jax.ShapeDtypeStruct((n_rows, embed), jnp.float32),
        scratch_types=[
            pltpu.VMEM((iw_per_w, _IW), jnp.int32),
            *([pltpu.VMEM((blk, embed), jnp.float32)] * _NBUF),
            *([pltpu.SemaphoreType.DMA] * (2 * _NBUF)),
        ],
    )
    def k(table_hbm, idx_hbm, out_hbm, idx_all, *bufs_and_sems):
        rows = bufs_and_sems[:_NBUF]
        sem_g = bufs_and_sems[_NBUF:2 * _NBUF]
        sem_o = bufs_and_sems[2 * _NBUF:]
        wid = lax.axis_index("s") * _NC + lax.axis_index("c")
        row_base = wid * n_per_w

        pltpu.sync_copy(
            idx_hbm.at[pl.ds(pl.multiple_of(wid * iw_per_w, 8), iw_per_w)],
            idx_all,
        )

        def fire_gathers(i, b):
            for t in range(_G):
                pltpu.async_copy(
                    table_hbm.at[idx_all.at[i * _G + t]],
                    rows[b].at[pl.ds(t * _IW, _IW)],
                    sem_g[b],
                )

        def drain_gathers(b):
            pltpu.make_async_copy(
                out_hbm.at[pl.ds(0, blk)], rows[b], sem_g[b]
            ).wait()

        def fire_store(i, b):
            row_off = pl.multiple_of(row_base + i * blk, 8)
            pltpu.async_copy(rows[b], out_hbm.at[pl.ds(row_off, blk)], sem_o[b])

        def drain_store(b):
            pltpu.make_async_copy(
                out_hbm.at[pl.ds(0, blk)], rows[b], sem_o[b]
            ).wait()

        def outer(jj, carry):
            for u in range(_NBUF):
                i = jj * _NBUF + u
                b = u

                @pl.when(jj > 0)
                def _():
                    drain_store(b)

                fire_gathers(i, b)

                b2 = (u - 2) % _NBUF
                if u >= 2:
                    drain_gathers(b2)
                    fire_store(i - 2, b2)
                else:
                    @pl.when(jj > 0)
                    def _():
                        drain_gathers(b2)
                        fire_store(i - 2, b2)
            return carry

        lax.fori_loop(0, n_blocks // _NBUF, outer, 0)

        for i in (n_blocks - 2, n_blocks - 1):
            b = i % _NBUF
            drain_gathers(b)
            fire_store(i, b)
        for b in range(_NBUF):
            drain_store(b)

    return k(table, idx2d)


def kernel(words, feats, table):
    batch, seq = words.shape
    vocab, embed = table.shape
    n_rows = batch * seq
    idx2d = words.reshape(n_rows // _IW, _IW)
    out = _gather_impl(table, idx2d, n_rows, embed)
    return out.reshape(batch, seq, embed)
